# baseline (speedup 1.0000x reference)
import functools

import jax
import jax.numpy as jnp
from jax import lax
from jax.experimental import pallas as pl
from jax.experimental.pallas import tpu as pltpu
from jax.experimental.pallas import tpu_sc as plsc

NC = 2
NS = 16
NW = NC * NS
L = 16


def _sc_kernel(B, DIM, CH):
  RPW = B // NW
  NCH = RPW // CH
  NBUF = 4
  assert RPW % CH == 0 and NCH % NBUF == 0 and CH % L == 0
  mesh = plsc.VectorSubcoreMesh(core_axis_name="c", subcore_axis_name="s")

  box_scratch = [pltpu.VMEM((CH, DIM), jnp.float32) for _ in range(8 * NBUF)]

  @functools.partial(
      pl.kernel,
      out_type=jax.ShapeDtypeStruct((B,), jnp.float32),
      mesh=mesh,
      scratch_types=[
          pltpu.VMEM((RPW,), jnp.int32),
          pltpu.VMEM((RPW,), jnp.int32),
          pltpu.VMEM((RPW,), jnp.int32),
          pltpu.VMEM((RPW,), jnp.int32),
          pltpu.VMEM((RPW,), jnp.float32),
          pltpu.VMEM((RPW,), jnp.float32),
      ] + box_scratch + [pltpu.SemaphoreType.DMA] * NBUF,
  )
  def body(min_hbm, max_hbm, rsc_hbm, rtr_hbm, ia_hbm, ib_hbm, ic_hbm,
           ir_hbm, f4_hbm, out_hbm, *scr):
    ia_v, ib_v, ic_v, ir_v, f4_v, out_v = scr[:6]
    bufs = [scr[6 + 8 * p:14 + 8 * p] for p in range(NBUF)]
    sems = scr[6 + 8 * NBUF:]
    tabs = [min_hbm, max_hbm, min_hbm, max_hbm, min_hbm, max_hbm,
            rsc_hbm, rtr_hbm]
    idxs = [ia_v, ia_v, ib_v, ib_v, ic_v, ic_v, ir_v, ir_v]

    wid = lax.axis_index("s") * NC + lax.axis_index("c")
    base = wid * RPW

    cps = [
        pltpu.async_copy(ia_hbm.at[pl.ds(base, RPW)], ia_v, sems[0]),
        pltpu.async_copy(ib_hbm.at[pl.ds(base, RPW)], ib_v, sems[0]),
        pltpu.async_copy(ic_hbm.at[pl.ds(base, RPW)], ic_v, sems[0]),
        pltpu.async_copy(ir_hbm.at[pl.ds(base, RPW)], ir_v, sems[0]),
        pltpu.async_copy(f4_hbm.at[pl.ds(base, RPW)], f4_v, sems[0]),
    ]
    for cp in cps:
      cp.wait()

    def issue(ch, p):
      for t in range(8):
        pltpu.async_copy(tabs[t].at[idxs[t].at[pl.ds(ch * CH, CH)]],
                         bufs[p][t], sems[p])

    def drain(p):
      for t in range(8):
        pltpu.make_async_copy(tabs[t].at[pl.ds(0, CH)], bufs[p][t],
                              sems[p]).wait()

    def compute(ch, p):
      minA, maxA, minB, maxB, minC, maxC, scv, trv = bufs[p]

      def grp_body(g, carry):
        off = ch * CH + g * L
        f4vec = f4_v[pl.ds(off, L)]
        lane = lax.iota(jnp.int32, L)
        lo8 = lane < 8
        x8 = lane ^ 8
        z16 = jnp.zeros((L,), jnp.int32)
        c8 = jnp.full((L,), 8, jnp.int32)

        def row_reduce(j, jf):
          z = jnp.zeros((L,), jnp.float32)
          f4b = f4vec.at[jnp.full((L,), jf, jnp.int32)].get(
              mode="promise_in_bounds")
          an, aden = z, z
          for c in range(DIM // L):
            sl = pl.ds(c * L, L)
            mA = minA[j, sl]
            MA = maxA[j, sl]
            mB = minB[j, sl]
            MB = maxB[j, sl]
            mC = minC[j, sl]
            MC = maxC[j, sl]
            sc = scv[j, sl]
            tr = trv[j, sl]
            mCp = mC * sc + tr
            MCp = MC * sc + tr
            m12 = jnp.maximum(mA, mB)
            M12 = jnp.minimum(MA, MB)
            mI = jnp.maximum(m12, mCp)
            MI = jnp.minimum(M12, MCp)
            dn = MI - mI
            d12 = M12 - m12
            dC = MCp - mCp
            dsel = d12 + f4b * (dC - d12)
            an = an + dn * dn
            aden = aden + dsel * dsel
          an = an + an.at[x8].get(mode="promise_in_bounds",
                                  unique_indices=True)
          aden = aden + aden.at[x8].get(mode="promise_in_bounds",
                                        unique_indices=True)
          m = jnp.where(lo8, an,
                        aden.at[x8].get(mode="promise_in_bounds",
                                        unique_indices=True))
          for s in (4, 2, 1):
            m = m + m.at[lane ^ s].get(mode="promise_in_bounds",
                                       unique_indices=True)
          nb = m.at[z16].get(mode="promise_in_bounds")
          db = m.at[c8].get(mode="promise_in_bounds")
          return nb / db

        def row_body(j2, res):
          j0 = g * L + 2 * j2
          jl = 2 * j2
          rv0 = row_reduce(j0, jl)
          rv1 = row_reduce(j0 + 1, jl + 1)
          res = jnp.where(lane == jl, rv0, res)
          res = jnp.where(lane == jl + 1, rv1, res)
          return res

        res = lax.fori_loop(0, L // 2, row_body,
                            jnp.zeros((L,), jnp.float32))
        out_v[pl.ds(off, L)] = res
        return carry

      lax.fori_loop(0, CH // L, grp_body, 0)

    for p in range(NBUF):
      issue(p, p)

    def round_body(i, carry):
      ch0 = NBUF * i
      for p in range(NBUF):
        drain(p)
        compute(ch0 + p, p)

        @pl.when(ch0 + p + NBUF < NCH)
        def _():
          issue(ch0 + p + NBUF, p)

      return carry

    lax.fori_loop(0, NCH // NBUF, round_body, 0)
    pltpu.sync_copy(out_v, out_hbm.at[pl.ds(base, RPW)])

  return body


def kernel(min_embeddings, max_embeddings, rel_scale_embeddings,
           rel_trans_embeddings, x):
  B = x.shape[0]
  DIM = min_embeddings.shape[1]
  REL = rel_scale_embeddings.shape[0]

  tag = x[:, 0]
  onehot = (tag[:, None] == jnp.arange(1, 5)[None, :]).astype(jnp.int32)
  ranks = jnp.cumsum(onehot, axis=0)
  offs = jnp.concatenate(
      [jnp.zeros((1,), jnp.int32), jnp.cumsum(ranks[-1])[:3]])
  pos = jnp.sum(onehot * (offs[None, :] + ranks - 1), axis=1)
  ts = tag
  c1, c2, c3 = x[:, 1], x[:, 2], x[:, 3]
  is12 = ts <= 2
  NID = 512
  ia = jnp.where(ts == 1, c2, c1)
  ib = jnp.where(is12, c2, c1)
  ic = jnp.where(is12, c3, c2)
  ir = jnp.where(is12, REL + (jnp.arange(B, dtype=jnp.int32) % NID), c3)
  f4 = (ts == 4).astype(jnp.float32)

  rsc = jnp.concatenate(
      [rel_scale_embeddings, jnp.ones((NID, DIM), jnp.float32)], axis=0)
  rtr = jnp.concatenate(
      [rel_trans_embeddings, jnp.zeros((NID, DIM), jnp.float32)], axis=0)

  res = _sc_kernel(B, DIM, CH=16)(
      min_embeddings, max_embeddings, rsc, rtr, ia, ib, ic, ir, f4)
  out = jnp.zeros((B,), jnp.float32).at[pos].set(
      res, mode="promise_in_bounds", unique_indices=True)
  return out[:, None]


# speedup vs baseline: 1.0449x; 1.0449x over previous
import functools

import jax
import jax.numpy as jnp
from jax import lax
from jax.experimental import pallas as pl
from jax.experimental.pallas import tpu as pltpu
from jax.experimental.pallas import tpu_sc as plsc

NC = 2
NS = 16
NW = NC * NS
L = 16


def _sc_kernel(B, DIM, CH):
  RPW = B // NW
  NCH = RPW // CH
  NBUF = 2
  assert RPW % CH == 0 and NCH % NBUF == 0 and CH % L == 0
  mesh = plsc.VectorSubcoreMesh(core_axis_name="c", subcore_axis_name="s")

  box_scratch = [pltpu.VMEM((CH, DIM), jnp.float32) for _ in range(8 * NBUF)]

  @functools.partial(
      pl.kernel,
      out_type=jax.ShapeDtypeStruct((B,), jnp.float32),
      mesh=mesh,
      scratch_types=[
          pltpu.VMEM((RPW,), jnp.int32),
          pltpu.VMEM((RPW,), jnp.int32),
          pltpu.VMEM((RPW,), jnp.int32),
          pltpu.VMEM((RPW,), jnp.int32),
          pltpu.VMEM((RPW,), jnp.float32),
          pltpu.VMEM((RPW,), jnp.float32),
      ] + box_scratch + [pltpu.SemaphoreType.DMA] * NBUF,
  )
  def body(min_hbm, max_hbm, rsc_hbm, rtr_hbm, ia_hbm, ib_hbm, ic_hbm,
           ir_hbm, f4_hbm, out_hbm, *scr):
    ia_v, ib_v, ic_v, ir_v, f4_v, out_v = scr[:6]
    bufs = [scr[6 + 8 * p:14 + 8 * p] for p in range(NBUF)]
    sems = scr[6 + 8 * NBUF:]
    tabs = [min_hbm, max_hbm, min_hbm, max_hbm, min_hbm, max_hbm,
            rsc_hbm, rtr_hbm]
    idxs = [ia_v, ia_v, ib_v, ib_v, ic_v, ic_v, ir_v, ir_v]

    wid = lax.axis_index("s") * NC + lax.axis_index("c")
    base = wid * RPW

    cps = [
        pltpu.async_copy(ia_hbm.at[pl.ds(base, RPW)], ia_v, sems[0]),
        pltpu.async_copy(ib_hbm.at[pl.ds(base, RPW)], ib_v, sems[0]),
        pltpu.async_copy(ic_hbm.at[pl.ds(base, RPW)], ic_v, sems[0]),
        pltpu.async_copy(ir_hbm.at[pl.ds(base, RPW)], ir_v, sems[0]),
        pltpu.async_copy(f4_hbm.at[pl.ds(base, RPW)], f4_v, sems[0]),
    ]
    for cp in cps:
      cp.wait()

    def issue(ch, p):
      for t in range(8):
        pltpu.async_copy(tabs[t].at[idxs[t].at[pl.ds(ch * CH, CH)]],
                         bufs[p][t], sems[p])

    def drain(p):
      for t in range(8):
        pltpu.make_async_copy(tabs[t].at[pl.ds(0, CH)], bufs[p][t],
                              sems[p]).wait()

    def compute(ch, p):
      minA, maxA, minB, maxB, minC, maxC, scv, trv = bufs[p]

      def grp_body(g, carry):
        off = ch * CH + g * L
        f4vec = f4_v[pl.ds(off, L)]
        lane = lax.iota(jnp.int32, L)
        lo8 = lane < 8
        x8 = lane ^ 8
        z16 = jnp.zeros((L,), jnp.int32)
        c8 = jnp.full((L,), 8, jnp.int32)

        def row_reduce(j, jf):
          z = jnp.zeros((L,), jnp.float32)
          f4b = f4vec.at[jnp.full((L,), jf, jnp.int32)].get(
              mode="promise_in_bounds")
          an, aden = z, z
          for c in range(DIM // L):
            sl = pl.ds(c * L, L)
            mA = minA[j, sl]
            MA = maxA[j, sl]
            mB = minB[j, sl]
            MB = maxB[j, sl]
            mC = minC[j, sl]
            MC = maxC[j, sl]
            sc = scv[j, sl]
            tr = trv[j, sl]
            mCp = mC * sc + tr
            MCp = MC * sc + tr
            m12 = jnp.maximum(mA, mB)
            M12 = jnp.minimum(MA, MB)
            mI = jnp.maximum(m12, mCp)
            MI = jnp.minimum(M12, MCp)
            dn = MI - mI
            d12 = M12 - m12
            dC = MCp - mCp
            dsel = d12 + f4b * (dC - d12)
            an = an + dn * dn
            aden = aden + dsel * dsel
          an = an + an.at[x8].get(mode="promise_in_bounds",
                                  unique_indices=True)
          aden = aden + aden.at[x8].get(mode="promise_in_bounds",
                                        unique_indices=True)
          m = jnp.where(lo8, an,
                        aden.at[x8].get(mode="promise_in_bounds",
                                        unique_indices=True))
          for s in (4, 2, 1):
            m = m + m.at[lane ^ s].get(mode="promise_in_bounds",
                                       unique_indices=True)
          nb = m.at[z16].get(mode="promise_in_bounds")
          db = m.at[c8].get(mode="promise_in_bounds")
          return nb / db

        def row_body(j2, res):
          j0 = g * L + 2 * j2
          jl = 2 * j2
          rv0 = row_reduce(j0, jl)
          rv1 = row_reduce(j0 + 1, jl + 1)
          res = jnp.where(lane == jl, rv0, res)
          res = jnp.where(lane == jl + 1, rv1, res)
          return res

        res = lax.fori_loop(0, L // 2, row_body,
                            jnp.zeros((L,), jnp.float32))
        out_v[pl.ds(off, L)] = res
        return carry

      lax.fori_loop(0, CH // L, grp_body, 0)

    for p in range(NBUF):
      issue(p, p)

    def round_body(i, carry):
      ch0 = NBUF * i
      for p in range(NBUF):
        drain(p)
        compute(ch0 + p, p)

        @pl.when(ch0 + p + NBUF < NCH)
        def _():
          issue(ch0 + p + NBUF, p)

      return carry

    lax.fori_loop(0, NCH // NBUF, round_body, 0)
    pltpu.sync_copy(out_v, out_hbm.at[pl.ds(base, RPW)])

  return body


def kernel(min_embeddings, max_embeddings, rel_scale_embeddings,
           rel_trans_embeddings, x):
  B = x.shape[0]
  DIM = min_embeddings.shape[1]
  REL = rel_scale_embeddings.shape[0]

  tag = x[:, 0]
  onehot = (tag[:, None] == jnp.arange(1, 5)[None, :]).astype(jnp.int32)
  ranks = jnp.cumsum(onehot, axis=0)
  offs = jnp.concatenate(
      [jnp.zeros((1,), jnp.int32), jnp.cumsum(ranks[-1])[:3]])
  pos = jnp.sum(onehot * (offs[None, :] + ranks - 1), axis=1)
  ts = tag
  c1, c2, c3 = x[:, 1], x[:, 2], x[:, 3]
  is12 = ts <= 2
  NID = 512
  ia = jnp.where(ts == 1, c2, c1)
  ib = jnp.where(is12, c2, c1)
  ic = jnp.where(is12, c3, c2)
  ir = jnp.where(is12, REL + (jnp.arange(B, dtype=jnp.int32) % NID), c3)
  f4 = (ts == 4).astype(jnp.float32)

  rsc = jnp.concatenate(
      [rel_scale_embeddings, jnp.ones((NID, DIM), jnp.float32)], axis=0)
  rtr = jnp.concatenate(
      [rel_trans_embeddings, jnp.zeros((NID, DIM), jnp.float32)], axis=0)

  res = _sc_kernel(B, DIM, CH=16)(
      min_embeddings, max_embeddings, rsc, rtr, ia, ib, ic, ir, f4)
  out = jnp.zeros((B,), jnp.float32).at[pos].set(
      res, mode="promise_in_bounds", unique_indices=True)
  return out[:, None]
